# baseline (speedup 1.0000x reference)
import functools

import jax
import jax.numpy as jnp
from jax import lax
from jax.experimental import pallas as pl
from jax.experimental.pallas import tpu as pltpu
from jax.experimental.pallas import tpu_sc as plsc

C, H, W = 19, 512, 512
HW = H * W
K = 100
KPAD = 112
D = 128
NC, NS = 2, 16
THRES = 0.1
NEG = -1.0


def _scores_topk_body(x_ref, ks_ref, svals_ref, scats_ref, kout_ref,
                      score_ref, bm_ref, rawv_ref, rawi_ref, rawp_ref,
                      stage_ref, dma_sem):
    c = pl.program_id(0)
    x = x_ref[0, 0]

    s = 1.0 / (1.0 + jnp.exp(-x))

    zcol = jnp.zeros((H, 1), jnp.float32)
    zrow = jnp.zeros((1, W), jnp.float32)
    rs = s + jnp.concatenate([s[:, 1:], zcol], axis=1) \
           + jnp.concatenate([zcol, s[:, :-1]], axis=1)
    sum9 = rs + jnp.concatenate([rs[1:, :], zrow], axis=0) \
              + jnp.concatenate([zrow, rs[:-1, :]], axis=0)
    cent = (s + sum9 * (1.0 / 9.0)) * 0.5

    ninf = jnp.float32(-jnp.inf)
    icol = jnp.full((H, 1), ninf)
    irow = jnp.full((1, W), ninf)
    mw = jnp.maximum(cent,
                     jnp.maximum(jnp.concatenate([cent[:, 1:], icol], axis=1),
                                 jnp.concatenate([icol, cent[:, :-1]], axis=1)))
    m3 = jnp.maximum(mw,
                     jnp.maximum(jnp.concatenate([mw[1:, :], irow], axis=0),
                                 jnp.concatenate([irow, mw[:-1, :]], axis=0)))
    score = jnp.where(m3 == cent, cent, 0.0)

    r0 = pl.multiple_of(c * H, H)
    score_ref[pl.ds(r0, H), :] = score

    s1 = jnp.max(score.reshape(H // 8, 8, W), axis=1)
    bm_c = jnp.max(s1, axis=1)
    bm_ref[pl.ds(c, 1)] = bm_c.reshape(1, H // 8)

    @pl.when(c == C - 1)
    def _extract():
        lane = lax.broadcasted_iota(jnp.int32, (1, 128), 1)
        rawv_ref[...] = jnp.zeros((1, 128), jnp.float32)
        rawi_ref[...] = jnp.zeros((1, 128), jnp.int32)

        bflat = (lax.broadcasted_iota(jnp.int32, (C, H // 8), 0) * (H // 8)
                 + lax.broadcasted_iota(jnp.int32, (C, H // 8), 1))
        ib = (lax.broadcasted_iota(jnp.int32, (8, W), 0) * W
              + lax.broadcasted_iota(jnp.int32, (8, W), 1))

        def body(i, carry):
            bm = bm_ref[...]
            m = jnp.max(bm)
            bsel = jnp.min(jnp.where(bm == m, bflat, jnp.int32(1 << 30)))
            tr0 = pl.multiple_of(bsel * 8, 8)
            blk = score_ref[pl.ds(tr0, 8), :]
            v = jnp.max(blk)
            pos = jnp.min(jnp.where(blk == v, ib, jnp.int32(1 << 30)))
            gidx = tr0 * W + pos
            blk_new = jnp.where(ib == pos, NEG, blk)
            score_ref[pl.ds(tr0, 8), :] = blk_new
            bm_ref[...] = jnp.where(bflat == bsel, jnp.max(blk_new), bm)
            rawv_ref[...] = jnp.where(lane == i, v, rawv_ref[...])
            rawi_ref[...] = jnp.where(lane == i, gidx, rawi_ref[...])
            row = tr0 + pos // W
            ph = row % H
            pw = pos % W
            pw0 = pl.multiple_of((pw // 128) * 128, 128)
            pltpu.make_async_copy(
                ks_ref.at[0, :, ph, pl.ds(pw0, 128)], stage_ref.at[:, i, :],
                dma_sem.at[lax.rem(i, 8)]).start()
            rawp_ref[...] = jnp.where(lane == i, pw % 128, rawp_ref[...])
            return carry

        lax.fori_loop(0, K, body, 0)

        def drain(i, carry):
            pltpu.make_async_copy(
                ks_ref.at[0, :, 0, pl.ds(0, 128)], stage_ref.at[:, 0, :],
                dma_sem.at[lax.rem(i, 8)]).wait()
            return carry

        lax.fori_loop(0, K, drain, 0)

        vals = rawv_ref[...]
        idx = rawi_ref[...]
        valid = lane < K
        keep = jnp.logical_and(vals > THRES, valid)
        svals_ref[...] = jnp.where(keep, vals, 0.0)
        scats_ref[...] = jnp.where(keep, idx // HW, 0)
        pwcol = jnp.transpose(rawp_ref[...])
        m8 = lax.broadcasted_iota(jnp.int32, (K, 128), 1) == pwcol[:K]
        st = stage_ref[...]
        sel = jnp.sum(jnp.where(m8[None], st, 0.0), axis=2)
        kout_ref[...] = jnp.transpose(sel)


def _detect(thing_map, kernel_space):
    return pl.pallas_call(
        _scores_topk_body,
        grid=(C,),
        in_specs=[
            pl.BlockSpec((1, 1, H, W), lambda c: (0, c, 0, 0)),
            pl.BlockSpec(memory_space=pltpu.MemorySpace.HBM),
        ],
        out_specs=[
            pl.BlockSpec((1, 128), lambda c: (0, 0)),
            pl.BlockSpec((1, 128), lambda c: (0, 0)),
            pl.BlockSpec((K, D), lambda c: (0, 0)),
        ],
        out_shape=[
            jax.ShapeDtypeStruct((1, 128), jnp.float32),
            jax.ShapeDtypeStruct((1, 128), jnp.int32),
            jax.ShapeDtypeStruct((K, D), jnp.float32),
        ],
        scratch_shapes=[
            pltpu.VMEM((C * H, W), jnp.float32),
            pltpu.VMEM((C, H // 8), jnp.float32),
            pltpu.VMEM((1, 128), jnp.float32),
            pltpu.VMEM((1, 128), jnp.int32),
            pltpu.VMEM((1, 128), jnp.int32),
            pltpu.VMEM((D, K, 128), jnp.float32),
            pltpu.SemaphoreType.DMA((8,)),
        ],
        compiler_params=pltpu.CompilerParams(
            dimension_semantics=("arbitrary",)),
    )(thing_map, kernel_space)


def kernel(thing_map, kernel_space):
    svals, scats, rows = _detect(thing_map, kernel_space)
    kernels = rows[None]
    scores = svals[:, :K]
    cats = scats[:, :K]
    return kernels, cats, scores


# speedup vs baseline: 1.0070x; 1.0070x over previous
import functools

import jax
import jax.numpy as jnp
from jax import lax
from jax.experimental import pallas as pl
from jax.experimental.pallas import tpu as pltpu
from jax.experimental.pallas import tpu_sc as plsc

C, H, W = 19, 512, 512
HW = H * W
K = 100
KPAD = 112
D = 128
NC, NS = 2, 16
THRES = 0.1
NEG = -1.0


def _scores_topk_body(x_ref, ks_ref, svals_ref, scats_ref, kout_ref,
                      score_ref, bm_ref, rawv_ref, rawi_ref, rawp_ref,
                      stage_ref, dma_sem):
    c = pl.program_id(0)
    x = x_ref[0, 0]

    s = 1.0 / (1.0 + jnp.exp(-x))

    zcol = jnp.zeros((H, 1), jnp.float32)
    zrow = jnp.zeros((1, W), jnp.float32)
    rs = s + jnp.concatenate([s[:, 1:], zcol], axis=1) \
           + jnp.concatenate([zcol, s[:, :-1]], axis=1)
    sum9 = rs + jnp.concatenate([rs[1:, :], zrow], axis=0) \
              + jnp.concatenate([zrow, rs[:-1, :]], axis=0)
    cent = (s + sum9 * (1.0 / 9.0)) * 0.5

    ninf = jnp.float32(-jnp.inf)
    icol = jnp.full((H, 1), ninf)
    irow = jnp.full((1, W), ninf)
    mw = jnp.maximum(cent,
                     jnp.maximum(jnp.concatenate([cent[:, 1:], icol], axis=1),
                                 jnp.concatenate([icol, cent[:, :-1]], axis=1)))
    m3 = jnp.maximum(mw,
                     jnp.maximum(jnp.concatenate([mw[1:, :], irow], axis=0),
                                 jnp.concatenate([irow, mw[:-1, :]], axis=0)))
    score = jnp.where(m3 == cent, cent, 0.0)

    r0 = pl.multiple_of(c * H, H)
    score_ref[pl.ds(r0, H), :] = score

    s1 = jnp.max(score.reshape(H // 8, 8, W), axis=1)
    bm_c = jnp.max(s1, axis=1)
    bm_ref[pl.ds(c, 1)] = bm_c.reshape(1, H // 8)

    @pl.when(c == C - 1)
    def _extract():
        lane = lax.broadcasted_iota(jnp.int32, (1, 128), 1)
        rawv_ref[...] = jnp.zeros((1, 128), jnp.float32)
        rawi_ref[...] = jnp.zeros((1, 128), jnp.int32)

        bflat = (lax.broadcasted_iota(jnp.int32, (C, H // 8), 0) * (H // 8)
                 + lax.broadcasted_iota(jnp.int32, (C, H // 8), 1))
        ib = (lax.broadcasted_iota(jnp.int32, (8, W), 0) * W
              + lax.broadcasted_iota(jnp.int32, (8, W), 1))

        def body(i, carry):
            bm, accv, acci, accp = carry
            m = jnp.max(bm)
            bsel = jnp.min(jnp.where(bm == m, bflat, jnp.int32(1 << 30)))
            tr0 = pl.multiple_of(bsel * 8, 8)
            blk = score_ref[pl.ds(tr0, 8), :]
            v = jnp.max(blk)
            pos = jnp.min(jnp.where(blk == v, ib, jnp.int32(1 << 30)))
            gidx = tr0 * W + pos
            blk_new = jnp.where(ib == pos, NEG, blk)
            score_ref[pl.ds(tr0, 8), :] = blk_new
            bm = jnp.where(bflat == bsel, jnp.max(blk_new), bm)
            sel = lane == i
            accv = jnp.where(sel, v, accv)
            acci = jnp.where(sel, gidx, acci)
            row = tr0 + pos // W
            ph = row % H
            pw = pos % W
            pw0 = pl.multiple_of((pw // 128) * 128, 128)
            pltpu.make_async_copy(
                ks_ref.at[0, :, ph, pl.ds(pw0, 128)], stage_ref.at[:, i, :],
                dma_sem).start()
            accp = jnp.where(sel, pw % 128, accp)
            return bm, accv, acci, accp

        z128f = jnp.zeros((1, 128), jnp.float32)
        z128i = jnp.zeros((1, 128), jnp.int32)
        _, fv, fi, fp = lax.fori_loop(
            0, K, body, (bm_ref[...], z128f, z128i, z128i))
        rawv_ref[...] = fv
        rawi_ref[...] = fi
        rawp_ref[...] = fp

        def drain(i, carry):
            pltpu.make_async_copy(
                ks_ref.at[0, :, 0, pl.ds(0, 128)], stage_ref.at[:, 0, :],
                dma_sem).wait()
            return carry

        lax.fori_loop(0, K, drain, 0)

        vals = rawv_ref[...]
        idx = rawi_ref[...]
        valid = lane < K
        keep = jnp.logical_and(vals > THRES, valid)
        svals_ref[...] = jnp.where(keep, vals, 0.0)
        scats_ref[...] = jnp.where(keep, idx // HW, 0)
        pwcol = jnp.transpose(rawp_ref[...])
        m8 = lax.broadcasted_iota(jnp.int32, (K, 128), 1) == pwcol[:K]
        st = stage_ref[...]
        sel = jnp.sum(jnp.where(m8[None], st, 0.0), axis=2)
        kout_ref[...] = jnp.transpose(sel)


def _detect(thing_map, kernel_space):
    return pl.pallas_call(
        _scores_topk_body,
        grid=(C,),
        in_specs=[
            pl.BlockSpec((1, 1, H, W), lambda c: (0, c, 0, 0)),
            pl.BlockSpec(memory_space=pltpu.MemorySpace.HBM),
        ],
        out_specs=[
            pl.BlockSpec((1, 128), lambda c: (0, 0)),
            pl.BlockSpec((1, 128), lambda c: (0, 0)),
            pl.BlockSpec((K, D), lambda c: (0, 0)),
        ],
        out_shape=[
            jax.ShapeDtypeStruct((1, 128), jnp.float32),
            jax.ShapeDtypeStruct((1, 128), jnp.int32),
            jax.ShapeDtypeStruct((K, D), jnp.float32),
        ],
        scratch_shapes=[
            pltpu.VMEM((C * H, W), jnp.float32),
            pltpu.VMEM((C, H // 8), jnp.float32),
            pltpu.VMEM((1, 128), jnp.float32),
            pltpu.VMEM((1, 128), jnp.int32),
            pltpu.VMEM((1, 128), jnp.int32),
            pltpu.VMEM((D, K, 128), jnp.float32),
            pltpu.SemaphoreType.DMA,
        ],
        compiler_params=pltpu.CompilerParams(
            dimension_semantics=("arbitrary",)),
    )(thing_map, kernel_space)


def kernel(thing_map, kernel_space):
    svals, scats, rows = _detect(thing_map, kernel_space)
    kernels = rows[None]
    scores = svals[:, :K]
    cats = scats[:, :K]
    return kernels, cats, scores


# speedup vs baseline: 1.0071x; 1.0001x over previous
import jax
import jax.numpy as jnp
from jax import lax
from jax.experimental import pallas as pl
from jax.experimental.pallas import tpu as pltpu

C, H, W = 19, 512, 512
HW = H * W
K = 100
D = 128
THRES = 0.1
NEG = -1.0


def _scores_topk_body(x_ref, ks_ref, svals_ref, scats_ref, kout_ref,
                      score_ref, bm_ref, rawv_ref, rawi_ref, rawp_ref,
                      stage_ref, dma_sem):
    c = pl.program_id(0)
    x = x_ref[0, 0]

    s = 1.0 / (1.0 + jnp.exp(-x))

    zcol = jnp.zeros((H, 1), jnp.float32)
    zrow = jnp.zeros((1, W), jnp.float32)
    rs = s + jnp.concatenate([s[:, 1:], zcol], axis=1) \
           + jnp.concatenate([zcol, s[:, :-1]], axis=1)
    sum9 = rs + jnp.concatenate([rs[1:, :], zrow], axis=0) \
              + jnp.concatenate([zrow, rs[:-1, :]], axis=0)
    cent = (s + sum9 * (1.0 / 9.0)) * 0.5

    ninf = jnp.float32(-jnp.inf)
    icol = jnp.full((H, 1), ninf)
    irow = jnp.full((1, W), ninf)
    mw = jnp.maximum(cent,
                     jnp.maximum(jnp.concatenate([cent[:, 1:], icol], axis=1),
                                 jnp.concatenate([icol, cent[:, :-1]], axis=1)))
    m3 = jnp.maximum(mw,
                     jnp.maximum(jnp.concatenate([mw[1:, :], irow], axis=0),
                                 jnp.concatenate([irow, mw[:-1, :]], axis=0)))
    score = jnp.where(m3 == cent, cent, 0.0)

    r0 = pl.multiple_of(c * H, H)
    score_ref[pl.ds(r0, H), :] = score

    s1 = jnp.max(score.reshape(H // 8, 8, W), axis=1)
    bm_c = jnp.max(s1, axis=1)
    bm_ref[pl.ds(c, 1)] = bm_c.reshape(1, H // 8)

    @pl.when(c == C - 1)
    def _extract():
        lane = lax.broadcasted_iota(jnp.int32, (1, 128), 1)
        rawv_ref[...] = jnp.zeros((1, 128), jnp.float32)
        rawi_ref[...] = jnp.zeros((1, 128), jnp.int32)

        bflat = (lax.broadcasted_iota(jnp.int32, (C, H // 8), 0) * (H // 8)
                 + lax.broadcasted_iota(jnp.int32, (C, H // 8), 1))
        ib = (lax.broadcasted_iota(jnp.int32, (8, W), 0) * W
              + lax.broadcasted_iota(jnp.int32, (8, W), 1))

        def body(i, carry):
            bm, accv, acci, accp = carry
            m = jnp.max(bm)
            bsel = jnp.min(jnp.where(bm == m, bflat, jnp.int32(1 << 30)))
            tr0 = pl.multiple_of(bsel * 8, 8)
            blk = score_ref[pl.ds(tr0, 8), :]
            v = jnp.max(blk)
            pos = jnp.min(jnp.where(blk == v, ib, jnp.int32(1 << 30)))
            gidx = tr0 * W + pos
            blk_new = jnp.where(ib == pos, NEG, blk)
            score_ref[pl.ds(tr0, 8), :] = blk_new
            bm = jnp.where(bflat == bsel, jnp.max(blk_new), bm)
            sel = lane == i
            accv = jnp.where(sel, v, accv)
            acci = jnp.where(sel, gidx, acci)
            row = tr0 + pos // W
            ph = row % H
            pw = pos % W
            pw0 = pl.multiple_of((pw // 128) * 128, 128)
            pltpu.make_async_copy(
                ks_ref.at[0, :, ph, pl.ds(pw0, 128)], stage_ref.at[:, i, :],
                dma_sem).start()
            accp = jnp.where(sel, pw % 128, accp)
            return bm, accv, acci, accp

        z128f = jnp.zeros((1, 128), jnp.float32)
        z128i = jnp.zeros((1, 128), jnp.int32)
        _, fv, fi, fp = lax.fori_loop(
            0, K, body, (bm_ref[...], z128f, z128i, z128i))
        rawv_ref[...] = fv
        rawi_ref[...] = fi
        rawp_ref[...] = fp

        def drain(i, carry):
            pltpu.make_async_copy(
                ks_ref.at[0, :, 0, pl.ds(0, 128)], stage_ref.at[:, 0, :],
                dma_sem).wait()
            return carry

        lax.fori_loop(0, K, drain, 0)

        vals = rawv_ref[...]
        idx = rawi_ref[...]
        valid = lane < K
        keep = jnp.logical_and(vals > THRES, valid)
        svals_ref[...] = jnp.where(keep, vals, 0.0)
        scats_ref[...] = jnp.where(keep, idx // HW, 0)
        pwcol = jnp.transpose(rawp_ref[...])
        m8 = lax.broadcasted_iota(jnp.int32, (K, 128), 1) == pwcol[:K]
        st = stage_ref[...]
        sel = jnp.sum(jnp.where(m8[None], st, 0.0), axis=2)
        kout_ref[...] = jnp.transpose(sel)


def _detect(thing_map, kernel_space):
    return pl.pallas_call(
        _scores_topk_body,
        grid=(C,),
        in_specs=[
            pl.BlockSpec((1, 1, H, W), lambda c: (0, c, 0, 0)),
            pl.BlockSpec(memory_space=pltpu.MemorySpace.HBM),
        ],
        out_specs=[
            pl.BlockSpec((1, 128), lambda c: (0, 0)),
            pl.BlockSpec((1, 128), lambda c: (0, 0)),
            pl.BlockSpec((K, D), lambda c: (0, 0)),
        ],
        out_shape=[
            jax.ShapeDtypeStruct((1, 128), jnp.float32),
            jax.ShapeDtypeStruct((1, 128), jnp.int32),
            jax.ShapeDtypeStruct((K, D), jnp.float32),
        ],
        scratch_shapes=[
            pltpu.VMEM((C * H, W), jnp.float32),
            pltpu.VMEM((C, H // 8), jnp.float32),
            pltpu.VMEM((1, 128), jnp.float32),
            pltpu.VMEM((1, 128), jnp.int32),
            pltpu.VMEM((1, 128), jnp.int32),
            pltpu.VMEM((D, K, 128), jnp.float32),
            pltpu.SemaphoreType.DMA,
        ],
        compiler_params=pltpu.CompilerParams(
            dimension_semantics=("arbitrary",)),
    )(thing_map, kernel_space)


def kernel(thing_map, kernel_space):
    svals, scats, rows = _detect(thing_map, kernel_space)
    kernels = rows[None]
    scores = svals[:, :K]
    cats = scats[:, :K]
    return kernels, cats, scores
